# Initial kernel scaffold; baseline (speedup 1.0000x reference)
import jax
import jax.numpy as jnp
from jax.experimental import pallas as pl


def kernel(hidden_states, gate_w, Wg, Wu, Wd):
    raise NotImplementedError("write your pallas kernel here")


# speedup vs baseline: 1.7024x; 1.7024x over previous
import jax
import jax.numpy as jnp
from jax.experimental import pallas as pl
from jax.experimental.pallas import tpu as pltpu

S, D, FF, E = 2048, 2048, 4096, 8
BLK = 512
NFF = 8
FFB = FF // NFF
NB = (2 * S) // BLK + E
P = NB * BLK


def _router_body(x_ref, gw_ref, e_ref, w_ref):
    x = x_ref[...]
    gw = gw_ref[...]
    logits = jax.lax.dot_general(
        x, gw, (((1,), (1,)), ((), ())), preferred_element_type=jnp.float32)
    idx = jax.lax.broadcasted_iota(jnp.int32, (S, E), 1)
    m1 = jnp.max(logits, axis=1, keepdims=True)
    e1 = jnp.min(jnp.where(logits == m1, idx, E), axis=1, keepdims=True)
    l2 = jnp.where(idx == e1, -jnp.inf, logits)
    m2 = jnp.max(l2, axis=1, keepdims=True)
    e2 = jnp.min(jnp.where(l2 == m2, idx, E), axis=1, keepdims=True)
    w1 = jax.nn.sigmoid(m1 - m2)
    e_ref[...] = jnp.concatenate([e1, e2], axis=1)
    w_ref[...] = jnp.concatenate([w1, 1.0 - w1], axis=1)


def _router(xf, gate_w):
    return pl.pallas_call(
        _router_body,
        out_shape=(
            jax.ShapeDtypeStruct((S, 2), jnp.int32),
            jax.ShapeDtypeStruct((S, 2), jnp.float32),
        ),
    )(xf, gate_w)


def _ffn_body(be_ref, act_ref, x_ref, wg_ref, wu_ref, wd_ref, w_ref, o_ref):
    nb = pl.program_id(0)
    ff = pl.program_id(1)

    @pl.when(act_ref[nb] == 1)
    def _():
        xb = x_ref[...]
        wg = wg_ref[0].astype(jnp.bfloat16)
        wu = wu_ref[0].astype(jnp.bfloat16)
        g = jax.lax.dot_general(
            xb, wg, (((1,), (1,)), ((), ())), preferred_element_type=jnp.float32)
        u = jax.lax.dot_general(
            xb, wu, (((1,), (1,)), ((), ())), preferred_element_type=jnp.float32)
        h = (g * jax.nn.sigmoid(g) * u).astype(jnp.bfloat16)
        wd = wd_ref[0].astype(jnp.bfloat16)
        y = jax.lax.dot_general(
            h, wd, (((1,), (1,)), ((), ())), preferred_element_type=jnp.float32)

        @pl.when(ff == 0)
        def _():
            o_ref[...] = y

        @pl.when(ff != 0)
        def _():
            o_ref[...] += y

        @pl.when(ff == NFF - 1)
        def _():
            o_ref[...] *= w_ref[...]


def _ffz(ff, a):
    return ff * a + (NFF - 1) * (1 - a)


def _grouped_ffn(be, act, xg16, Wg, Wu, Wd, wpad2):
    grid_spec = pltpu.PrefetchScalarGridSpec(
        num_scalar_prefetch=2,
        grid=(NB, NFF),
        in_specs=[
            pl.BlockSpec((BLK, D), lambda nb, ff, be, act: (nb, 0)),
            pl.BlockSpec((1, FFB, D),
                         lambda nb, ff, be, act: (be[nb], _ffz(ff, act[nb]), 0)),
            pl.BlockSpec((1, FFB, D),
                         lambda nb, ff, be, act: (be[nb], _ffz(ff, act[nb]), 0)),
            pl.BlockSpec((1, D, FFB),
                         lambda nb, ff, be, act: (be[nb], 0, _ffz(ff, act[nb]))),
            pl.BlockSpec((BLK, 1), lambda nb, ff, be, act: (nb, 0)),
        ],
        out_specs=pl.BlockSpec((BLK, D), lambda nb, ff, be, act: (nb, 0)),
    )
    return pl.pallas_call(
        _ffn_body,
        grid_spec=grid_spec,
        out_shape=jax.ShapeDtypeStruct((P, D), jnp.float32),
        compiler_params=pltpu.CompilerParams(
            dimension_semantics=("parallel", "arbitrary")),
    )(be, act, xg16, Wg, Wu, Wd, wpad2)


def kernel(hidden_states, gate_w, Wg, Wu, Wd):
    b, s, d = hidden_states.shape
    xf = hidden_states.reshape(S, D)

    e_out, w_out = _router(xf, gate_w)

    ef = e_out.reshape(-1)
    wf = w_out.reshape(-1)
    oh = (ef[:, None] == jnp.arange(E, dtype=jnp.int32)[None, :]).astype(jnp.int32)
    cs = jnp.cumsum(oh, axis=0)
    rank = jnp.take_along_axis(cs, ef[:, None], axis=1)[:, 0] - 1
    counts = cs[-1]
    pc = ((counts + BLK - 1) // BLK) * BLK
    ends = jnp.cumsum(pc)
    off = ends - pc
    p = off[ef] + rank
    tok = jnp.arange(2 * S, dtype=jnp.int32) // 2
    gidx = jnp.zeros((P,), jnp.int32).at[p].set(tok)
    wpad = jnp.zeros((P,), jnp.float32).at[p].set(wf)

    nb_start = jnp.arange(NB, dtype=jnp.int32) * BLK
    total = ends[-1]
    act = (nb_start < total).astype(jnp.int32)
    be_raw = jnp.sum((ends[None, :] <= nb_start[:, None]).astype(jnp.int32), axis=1)
    last_e = be_raw[(total // BLK) - 1]
    be = jnp.where(act == 1, be_raw, last_e).astype(jnp.int32)

    x16 = xf.astype(jnp.bfloat16)
    xg16 = jnp.take(x16, gidx, axis=0)

    y = _grouped_ffn(be, act, xg16, Wg, Wu, Wd, wpad.reshape(P, 1))

    p1 = p[0::2]
    p2 = p[1::2]
    out = jnp.take(y, p1, axis=0) + jnp.take(y, p2, axis=0)
    return out.reshape(b, s, d)
